# baseline (device time: 45337 ns/iter reference)
import jax
import jax.numpy as jnp
from jax import lax
from jax.experimental import pallas as pl
from jax.experimental.pallas import tpu as pltpu

B, SQ, H, D = 2, 256, 8, 64
SCALE = D ** -0.5


def kernel(Q, K, V):
    def body(q_ref, k_ref, v_ref, out_ref, kv_send, kv_recv, send_sem, recv_sem):
        my_x = lax.axis_index("x")
        my_y = lax.axis_index("y")
        my_z = lax.axis_index("z")
        nbr = (my_x, 1 - my_y, my_z)

        barrier_sem = pltpu.get_barrier_semaphore()
        pl.semaphore_signal(
            barrier_sem, inc=1, device_id=nbr,
            device_id_type=pl.DeviceIdType.MESH,
        )
        pl.semaphore_wait(barrier_sem, 1)

        kv_send[0] = k_ref[...].astype(jnp.bfloat16)
        kv_send[1] = v_ref[...].astype(jnp.bfloat16)
        rdma = pltpu.make_async_remote_copy(
            src_ref=kv_send,
            dst_ref=kv_recv,
            send_sem=send_sem,
            recv_sem=recv_sem,
            device_id=nbr,
            device_id_type=pl.DeviceIdType.MESH,
        )
        rdma.start()
        rdma.wait()

        dn_t = (((1,), (1,)), ((), ()))
        for b in range(B):
            for h in range(H):
                q = q_ref[b, :, h, :].astype(jnp.bfloat16)
                kl = kv_send[0, b, :, h, :]
                kr = kv_recv[0, b, :, h, :]
                vl = kv_send[1, b, :, h, :]
                vr = kv_recv[1, b, :, h, :]
                s1 = lax.dot_general(q, kl, dn_t,
                                     preferred_element_type=jnp.float32) * SCALE
                s2 = lax.dot_general(q, kr, dn_t,
                                     preferred_element_type=jnp.float32) * SCALE
                m = jnp.maximum(s1.max(-1, keepdims=True),
                                s2.max(-1, keepdims=True))
                p1 = jnp.exp(s1 - m)
                p2 = jnp.exp(s2 - m)
                l = p1.sum(-1, keepdims=True) + p2.sum(-1, keepdims=True)
                o = lax.dot_general(p1.astype(jnp.bfloat16), vl,
                                    (((1,), (0,)), ((), ())),
                                    preferred_element_type=jnp.float32)
                o = o + lax.dot_general(p2.astype(jnp.bfloat16), vr,
                                        (((1,), (0,)), ((), ())),
                                        preferred_element_type=jnp.float32)
                out_ref[b, :, h, :] = o / l

    return pl.pallas_call(
        body,
        out_shape=jax.ShapeDtypeStruct((B, SQ, H, D), jnp.float32),
        in_specs=[pl.BlockSpec(memory_space=pltpu.VMEM)] * 3,
        out_specs=pl.BlockSpec(memory_space=pltpu.VMEM),
        scratch_shapes=[
            pltpu.VMEM((2, B, SQ, H, D), jnp.bfloat16),
            pltpu.VMEM((2, B, SQ, H, D), jnp.bfloat16),
            pltpu.SemaphoreType.DMA,
            pltpu.SemaphoreType.DMA,
        ],
        compiler_params=pltpu.CompilerParams(collective_id=0),
    )(Q, K, V)


# device time: 30517 ns/iter; 1.4856x vs baseline; 1.4856x over previous
import jax
import jax.numpy as jnp
from jax import lax
from jax.experimental import pallas as pl
from jax.experimental.pallas import tpu as pltpu

B, SQ, H, D = 2, 256, 8, 64
SCALE = D ** -0.5


def kernel(Q, K, V):
    def body(q_ref, k_ref, v_ref, out_ref, kv_send, kv_recv, send_sem, recv_sem):
        my_x = lax.axis_index("x")
        my_y = lax.axis_index("y")
        my_z = lax.axis_index("z")
        nbr = (my_x, 1 - my_y, my_z)

        barrier_sem = pltpu.get_barrier_semaphore()
        pl.semaphore_signal(
            barrier_sem, inc=1, device_id=nbr,
            device_id_type=pl.DeviceIdType.MESH,
        )
        pl.semaphore_wait(barrier_sem, 1)

        for b in range(B):
            for h in range(H):
                kv_send[0, b, h] = k_ref[b, :, h, :].astype(jnp.bfloat16).T
                kv_send[1, b, h] = v_ref[b, :, h, :].astype(jnp.bfloat16).T
        rdma = pltpu.make_async_remote_copy(
            src_ref=kv_send,
            dst_ref=kv_recv,
            send_sem=send_sem,
            recv_sem=recv_sem,
            device_id=nbr,
            device_id_type=pl.DeviceIdType.MESH,
        )
        rdma.start()
        rdma.wait()

        dn_nn = (((1,), (0,)), ((), ()))
        dn_nt = (((1,), (1,)), ((), ()))
        for b in range(B):
            for h in range(H):
                q = q_ref[b, :, h, :].astype(jnp.bfloat16)
                klT = kv_send[0, b, h]
                krT = kv_recv[0, b, h]
                vlT = kv_send[1, b, h]
                vrT = kv_recv[1, b, h]
                s1 = lax.dot_general(q, klT, dn_nn,
                                     preferred_element_type=jnp.float32) * SCALE
                s2 = lax.dot_general(q, krT, dn_nn,
                                     preferred_element_type=jnp.float32) * SCALE
                m = jnp.maximum(s1.max(-1, keepdims=True),
                                s2.max(-1, keepdims=True))
                p1 = jnp.exp(s1 - m)
                p2 = jnp.exp(s2 - m)
                l = p1.sum(-1, keepdims=True) + p2.sum(-1, keepdims=True)
                o = lax.dot_general(p1.astype(jnp.bfloat16), vlT, dn_nt,
                                    preferred_element_type=jnp.float32)
                o = o + lax.dot_general(p2.astype(jnp.bfloat16), vrT, dn_nt,
                                        preferred_element_type=jnp.float32)
                out_ref[b, :, h, :] = o / l

    return pl.pallas_call(
        body,
        out_shape=jax.ShapeDtypeStruct((B, SQ, H, D), jnp.float32),
        in_specs=[pl.BlockSpec(memory_space=pltpu.VMEM)] * 3,
        out_specs=pl.BlockSpec(memory_space=pltpu.VMEM),
        scratch_shapes=[
            pltpu.VMEM((2, B, H, D, SQ), jnp.bfloat16),
            pltpu.VMEM((2, B, H, D, SQ), jnp.bfloat16),
            pltpu.SemaphoreType.DMA,
            pltpu.SemaphoreType.DMA,
        ],
        compiler_params=pltpu.CompilerParams(collective_id=0),
    )(Q, K, V)


# device time: 27203 ns/iter; 1.6666x vs baseline; 1.1218x over previous
import jax
import jax.numpy as jnp
from jax import lax
from jax.experimental import pallas as pl
from jax.experimental.pallas import tpu as pltpu

B, SQ, H, D = 2, 256, 8, 64
SCALE = D ** -0.5


def _comm(km, vm):

    def body(k_ref, v_ref, loc_ref, rcv_ref, send_sem, recv_sem):
        my_x = lax.axis_index("x")
        my_y = lax.axis_index("y")
        my_z = lax.axis_index("z")
        nbr = (my_x, 1 - my_y, my_z)

        barrier_sem = pltpu.get_barrier_semaphore()
        pl.semaphore_signal(
            barrier_sem, inc=1, device_id=nbr,
            device_id_type=pl.DeviceIdType.MESH,
        )
        pl.semaphore_wait(barrier_sem, 1)

        loc_ref[0] = k_ref[...].astype(jnp.bfloat16)
        loc_ref[1] = v_ref[...].astype(jnp.bfloat16)
        rdma = pltpu.make_async_remote_copy(
            src_ref=loc_ref,
            dst_ref=rcv_ref,
            send_sem=send_sem,
            recv_sem=recv_sem,
            device_id=nbr,
            device_id_type=pl.DeviceIdType.MESH,
        )
        rdma.start()
        rdma.wait()

    return pl.pallas_call(
        body,
        out_shape=(
            jax.ShapeDtypeStruct((2, B, SQ, H * D), jnp.bfloat16),
            jax.ShapeDtypeStruct((2, B, SQ, H * D), jnp.bfloat16),
        ),
        in_specs=[pl.BlockSpec(memory_space=pltpu.VMEM)] * 2,
        out_specs=(
            pl.BlockSpec(memory_space=pltpu.VMEM),
            pl.BlockSpec(memory_space=pltpu.VMEM),
        ),
        scratch_shapes=[
            pltpu.SemaphoreType.DMA,
            pltpu.SemaphoreType.DMA,
        ],
        compiler_params=pltpu.CompilerParams(collective_id=0),
    )(km, vm)


def _attn(qm, kv_local, kv_recv):
    dn_nt = (((1,), (1,)), ((), ()))
    dn_nn = (((1,), (0,)), ((), ()))

    def one_head(q, kl, vl, kr, vr):
        s1 = lax.dot_general(q, kl, dn_nt,
                             preferred_element_type=jnp.float32) * SCALE
        s2 = lax.dot_general(q, kr, dn_nt,
                             preferred_element_type=jnp.float32) * SCALE
        m = jnp.maximum(s1.max(-1, keepdims=True), s2.max(-1, keepdims=True))
        p1 = jnp.exp(s1 - m)
        p2 = jnp.exp(s2 - m)
        l = p1.sum(-1, keepdims=True) + p2.sum(-1, keepdims=True)
        o = lax.dot_general(p1.astype(jnp.bfloat16), vl, dn_nn,
                            preferred_element_type=jnp.float32)
        o = o + lax.dot_general(p2.astype(jnp.bfloat16), vr, dn_nn,
                                preferred_element_type=jnp.float32)
        return o / l

    def body(q_ref, kvl_ref, kvr_ref, out_ref):
        q2 = q_ref[0].astype(jnp.bfloat16)
        kl2 = kvl_ref[0, 0]
        vl2 = kvl_ref[1, 0]
        kr2 = kvr_ref[0, 0]
        vr2 = kvr_ref[1, 0]
        oa = one_head(q2[:, :D], kl2[:, :D], vl2[:, :D],
                      kr2[:, :D], vr2[:, :D])
        ob = one_head(q2[:, D:], kl2[:, D:], vl2[:, D:],
                      kr2[:, D:], vr2[:, D:])
        out_ref[0] = jnp.concatenate([oa, ob], axis=1)

    return pl.pallas_call(
        body,
        grid=(B, H // 2),
        out_shape=jax.ShapeDtypeStruct((B, SQ, H * D), jnp.float32),
        in_specs=[
            pl.BlockSpec((1, SQ, 2 * D), lambda b, hh: (b, 0, hh)),
            pl.BlockSpec((2, 1, SQ, 2 * D), lambda b, hh: (0, b, 0, hh)),
            pl.BlockSpec((2, 1, SQ, 2 * D), lambda b, hh: (0, b, 0, hh)),
        ],
        out_specs=pl.BlockSpec((1, SQ, 2 * D), lambda b, hh: (b, 0, hh)),
    )(qm, kv_local, kv_recv)


def kernel(Q, K, V):
    km = K.reshape(B, SQ, H * D)
    vm = V.reshape(B, SQ, H * D)
    qm = Q.reshape(B, SQ, H * D)
    kv_local, kv_recv = _comm(km, vm)
    om = _attn(qm, kv_local, kv_recv)
    return om.reshape(B, SQ, H, D)


# device time: 20834 ns/iter; 2.1761x vs baseline; 1.3057x over previous
import jax
import jax.numpy as jnp
from jax import lax
from jax.experimental import pallas as pl
from jax.experimental.pallas import tpu as pltpu

B, SQ, H, D = 2, 256, 8, 64
HD = H * D
SCALE = D ** -0.5

dn_nt = (((1,), (1,)), ((), ()))
dn_nn = (((1,), (0,)), ((), ()))


def _fused(q4, km, vm):
    def body(q_ref, km_ref, vm_ref, om_ref, loc, rcv, ysend, yrecv):
        my_x = lax.axis_index("x")
        my_y = lax.axis_index("y")
        my_z = lax.axis_index("z")
        ypeer = (my_x, 1 - my_y, my_z)

        barrier_sem = pltpu.get_barrier_semaphore()
        pl.semaphore_signal(
            barrier_sem, inc=1, device_id=ypeer,
            device_id_type=pl.DeviceIdType.MESH,
        )
        pl.semaphore_wait(barrier_sem, 1)

        rdmas = []
        for b in range(B):
            loc[b, :, :HD] = km_ref[b].astype(jnp.bfloat16)
            loc[b, :, HD:] = vm_ref[b].astype(jnp.bfloat16)
            r = pltpu.make_async_remote_copy(
                src_ref=loc.at[b],
                dst_ref=rcv.at[b],
                send_sem=ysend.at[b],
                recv_sem=yrecv.at[b],
                device_id=ypeer,
                device_id_type=pl.DeviceIdType.MESH,
            )
            r.start()
            rdmas.append(r)

        part = []
        for b in range(B):
            row = []
            for hh in range(H // 2):
                q2 = jnp.concatenate(
                    [q_ref[b, :, 2 * hh, :], q_ref[b, :, 2 * hh + 1, :]],
                    axis=1,
                ).astype(jnp.bfloat16)
                kl2 = loc[b, :, hh * 2 * D:(hh + 1) * 2 * D]
                vl2 = loc[b, :, HD + hh * 2 * D:HD + (hh + 1) * 2 * D]
                for i in range(2):
                    q = q2[:, i * D:(i + 1) * D]
                    kl = kl2[:, i * D:(i + 1) * D]
                    vl = vl2[:, i * D:(i + 1) * D]
                    s1 = lax.dot_general(
                        q, kl, dn_nt, preferred_element_type=jnp.float32
                    ) * SCALE
                    m1 = s1.max(-1, keepdims=True)
                    p1 = jnp.exp(s1 - m1)
                    l1 = p1.sum(-1, keepdims=True)
                    o1 = lax.dot_general(
                        p1.astype(jnp.bfloat16), vl, dn_nn,
                        preferred_element_type=jnp.float32,
                    )
                    row.append((q, o1, m1, l1))
            part.append(row)

        for b in range(B):
            rdmas[b].wait_recv()
            for hh in range(H // 2):
                kr2 = rcv[b, :, hh * 2 * D:(hh + 1) * 2 * D]
                vr2 = rcv[b, :, HD + hh * 2 * D:HD + (hh + 1) * 2 * D]
                outs = []
                for i in range(2):
                    q, o1, m1, l1 = part[b][2 * hh + i]
                    kr = kr2[:, i * D:(i + 1) * D]
                    vr = vr2[:, i * D:(i + 1) * D]
                    s2 = lax.dot_general(
                        q, kr, dn_nt, preferred_element_type=jnp.float32
                    ) * SCALE
                    m2 = s2.max(-1, keepdims=True)
                    m = jnp.maximum(m1, m2)
                    p2 = jnp.exp(s2 - m)
                    o2 = lax.dot_general(
                        p2.astype(jnp.bfloat16), vr, dn_nn,
                        preferred_element_type=jnp.float32,
                    )
                    alpha = jnp.exp(m1 - m)
                    o = o1 * alpha + o2
                    l = l1 * alpha + p2.sum(-1, keepdims=True)
                    outs.append(o / l)
                om_ref[b, :, hh * 2 * D:(hh + 1) * 2 * D] = jnp.concatenate(
                    outs, axis=1
                )

        for b in range(B):
            rdmas[b].wait_send()

    return pl.pallas_call(
        body,
        out_shape=jax.ShapeDtypeStruct((B, SQ, HD), jnp.float32),
        in_specs=[pl.BlockSpec(memory_space=pltpu.VMEM)] * 3,
        out_specs=pl.BlockSpec(memory_space=pltpu.VMEM),
        scratch_shapes=[
            pltpu.VMEM((B, SQ, 2 * HD), jnp.bfloat16),
            pltpu.VMEM((B, SQ, 2 * HD), jnp.bfloat16),
            pltpu.SemaphoreType.DMA((B,)),
            pltpu.SemaphoreType.DMA((B,)),
        ],
        compiler_params=pltpu.CompilerParams(collective_id=0),
    )(q4, km, vm)


def kernel(Q, K, V):
    km = K.reshape(B, SQ, HD)
    vm = V.reshape(B, SQ, HD)
    om = _fused(Q, km, vm)
    return om.reshape(B, SQ, H, D)


# device time: 19415 ns/iter; 2.3352x vs baseline; 1.0731x over previous
import jax
import jax.numpy as jnp
from jax import lax
from jax.experimental import pallas as pl
from jax.experimental.pallas import tpu as pltpu

B, SQ, H, D = 2, 256, 8, 64
HD = H * D
SCALE = D ** -0.5
C = 8
ROWS = SQ // C

dn_nt = (((1,), (1,)), ((), ()))
dn_nn = (((1,), (0,)), ((), ()))


def _fused(q4, km, vm):
    def body(q_ref, km_ref, vm_ref, om_ref, loc, rcv,
             ysend, yrecv, fsend, frecv):
        my_x = lax.axis_index("x")
        my_y = lax.axis_index("y")
        my_z = lax.axis_index("z")

        barrier_sem = pltpu.get_barrier_semaphore()
        for dev in ((my_x, 1 - my_y, my_z), (1 - my_x, my_y, my_z)):
            pl.semaphore_signal(
                barrier_sem, inc=1, device_id=dev,
                device_id_type=pl.DeviceIdType.MESH,
            )
        pl.semaphore_wait(barrier_sem, 2)

        def pack(b):
            loc[b, :, :HD] = km_ref[b].astype(jnp.bfloat16)
            loc[b, :, HD:] = vm_ref[b].astype(jnp.bfloat16)

        def local_unit(b, hh):
            q2 = jnp.concatenate(
                [q_ref[b, :, 2 * hh, :], q_ref[b, :, 2 * hh + 1, :]],
                axis=1,
            ).astype(jnp.bfloat16)
            kl2 = loc[b, :, hh * 2 * D:(hh + 1) * 2 * D]
            vl2 = loc[b, :, HD + hh * 2 * D:HD + (hh + 1) * 2 * D]
            units = []
            for i in range(2):
                q = q2[:, i * D:(i + 1) * D]
                kl = kl2[:, i * D:(i + 1) * D]
                vl = vl2[:, i * D:(i + 1) * D]
                s1 = lax.dot_general(
                    q, kl, dn_nt, preferred_element_type=jnp.float32
                ) * SCALE
                m1 = s1.max(-1, keepdims=True)
                p1 = jnp.exp(s1 - m1)
                l1 = p1.sum(-1, keepdims=True)
                o1 = lax.dot_general(
                    p1.astype(jnp.bfloat16), vl, dn_nn,
                    preferred_element_type=jnp.float32,
                )
                units.append((q, o1, m1, l1))
            return units

        def remote_batch(b, part):
            for hh in range(H // 2):
                kr2 = rcv[b, :, hh * 2 * D:(hh + 1) * 2 * D]
                vr2 = rcv[b, :, HD + hh * 2 * D:HD + (hh + 1) * 2 * D]
                outs = []
                for i in range(2):
                    q, o1, m1, l1 = part[2 * hh + i]
                    kr = kr2[:, i * D:(i + 1) * D]
                    vr = vr2[:, i * D:(i + 1) * D]
                    s2 = lax.dot_general(
                        q, kr, dn_nt, preferred_element_type=jnp.float32
                    ) * SCALE
                    m = jnp.maximum(m1, s2.max(-1, keepdims=True))
                    p2 = jnp.exp(s2 - m)
                    o2 = lax.dot_general(
                        p2.astype(jnp.bfloat16), vr, dn_nn,
                        preferred_element_type=jnp.float32,
                    )
                    alpha = jnp.exp(m1 - m)
                    o = o1 * alpha + o2
                    l = l1 * alpha + p2.sum(-1, keepdims=True)
                    outs.append(o / l)
                om_ref[b, :, hh * 2 * D:(hh + 1) * 2 * D] = jnp.concatenate(
                    outs, axis=1
                )

        def run(mx):
            ox = 1 - mx
            ypeer = (mx, 1 - my_y, my_z)
            xtwin = (ox, my_y, my_z)

            pack(mx)
            ych = []
            for c in range(C):
                rows = pl.ds(c * ROWS, ROWS)
                r = pltpu.make_async_remote_copy(
                    src_ref=loc.at[mx, rows, :],
                    dst_ref=rcv.at[mx, rows, :],
                    send_sem=ysend.at[c],
                    recv_sem=yrecv.at[c],
                    device_id=ypeer,
                    device_id_type=pl.DeviceIdType.MESH,
                )
                r.start()
                ych.append(r)
            pack(ox)

            fch = [
                pltpu.make_async_remote_copy(
                    src_ref=rcv.at[mx, pl.ds(c * ROWS, ROWS), :],
                    dst_ref=rcv.at[mx, pl.ds(c * ROWS, ROWS), :],
                    send_sem=fsend.at[c],
                    recv_sem=frecv.at[c],
                    device_id=xtwin,
                    device_id_type=pl.DeviceIdType.MESH,
                )
                for c in range(C)
            ]

            units = [(b, hh) for b in range(B) for hh in range(H // 2)]
            part = {0: [], 1: []}
            for c in range(C):
                ych[c].wait_recv()
                fch[c].start()
                b, hh = units[c]
                part[b].extend(local_unit(b, hh))

            remote_batch(mx, part[mx])

            for c in range(C):
                fwd_in = pltpu.make_async_remote_copy(
                    src_ref=rcv.at[ox, pl.ds(c * ROWS, ROWS), :],
                    dst_ref=rcv.at[ox, pl.ds(c * ROWS, ROWS), :],
                    send_sem=ysend.at[c],
                    recv_sem=frecv.at[c],
                    device_id=xtwin,
                    device_id_type=pl.DeviceIdType.MESH,
                )
                fwd_in.wait_recv()
            remote_batch(ox, part[ox])

            for r in ych:
                r.wait_send()
            for r in fch:
                r.wait_send()

        @pl.when(my_x == 0)
        def _():
            run(0)

        @pl.when(my_x == 1)
        def _():
            run(1)

    return pl.pallas_call(
        body,
        out_shape=jax.ShapeDtypeStruct((B, SQ, HD), jnp.float32),
        in_specs=[pl.BlockSpec(memory_space=pltpu.VMEM)] * 3,
        out_specs=pl.BlockSpec(memory_space=pltpu.VMEM),
        scratch_shapes=[
            pltpu.VMEM((B, SQ, 2 * HD), jnp.bfloat16),
            pltpu.VMEM((B, SQ, 2 * HD), jnp.bfloat16),
            pltpu.SemaphoreType.DMA((C,)),
            pltpu.SemaphoreType.DMA((C,)),
            pltpu.SemaphoreType.DMA((C,)),
            pltpu.SemaphoreType.DMA((C,)),
        ],
        compiler_params=pltpu.CompilerParams(collective_id=0),
    )(q4, km, vm)


def kernel(Q, K, V):
    km = K.reshape(B, SQ, HD)
    vm = V.reshape(B, SQ, HD)
    om = _fused(Q, km, vm)
    return om.reshape(B, SQ, H, D)


# device time: 19220 ns/iter; 2.3588x vs baseline; 1.0101x over previous
import jax
import jax.numpy as jnp
from jax import lax
from jax.experimental import pallas as pl
from jax.experimental.pallas import tpu as pltpu

B, SQ, H, D = 2, 256, 8, 64
HD = H * D
SCALE = D ** -0.5
C = 8
ROWS = SQ // C

dn_nt = (((1,), (1,)), ((), ()))
dn_nn = (((1,), (0,)), ((), ()))


def _fused(qt, km, vm):
    def body(q_ref, km_ref, vm_ref, om_ref, loc, rcv,
             ysend, yrecv, fsend, frecv):
        my_x = lax.axis_index("x")
        my_y = lax.axis_index("y")
        my_z = lax.axis_index("z")

        barrier_sem = pltpu.get_barrier_semaphore()
        for dev in ((my_x, 1 - my_y, my_z), (1 - my_x, my_y, my_z)):
            pl.semaphore_signal(
                barrier_sem, inc=1, device_id=dev,
                device_id_type=pl.DeviceIdType.MESH,
            )
        pl.semaphore_wait(barrier_sem, 2)

        def pack(b):
            loc[b, :, :HD] = km_ref[b].astype(jnp.bfloat16)
            loc[b, :, HD:] = vm_ref[b].astype(jnp.bfloat16)

        def local_unit(b, hh):
            kl2 = loc[b, :, hh * 2 * D:(hh + 1) * 2 * D]
            vl2 = loc[b, :, HD + hh * 2 * D:HD + (hh + 1) * 2 * D]
            units = []
            for i in range(2):
                h = 2 * hh + i
                q = q_ref[b, h].astype(jnp.bfloat16)
                kl = kl2[:, i * D:(i + 1) * D]
                vl = vl2[:, i * D:(i + 1) * D]
                s1 = lax.dot_general(
                    q, kl, dn_nt, preferred_element_type=jnp.float32
                ) * SCALE
                p1 = jnp.exp(s1)
                l1 = p1.sum(-1, keepdims=True)
                o1 = lax.dot_general(
                    p1.astype(jnp.bfloat16), vl, dn_nn,
                    preferred_element_type=jnp.float32,
                )
                units.append((q, o1, l1))
            return units

        def remote_batch(b, part):
            for hh in range(H // 2):
                kr2 = rcv[b, :, hh * 2 * D:(hh + 1) * 2 * D]
                vr2 = rcv[b, :, HD + hh * 2 * D:HD + (hh + 1) * 2 * D]
                for i in range(2):
                    q, o1, l1 = part[2 * hh + i]
                    kr = kr2[:, i * D:(i + 1) * D]
                    vr = vr2[:, i * D:(i + 1) * D]
                    s2 = lax.dot_general(
                        q, kr, dn_nt, preferred_element_type=jnp.float32
                    ) * SCALE
                    p2 = jnp.exp(s2)
                    o2 = lax.dot_general(
                        p2.astype(jnp.bfloat16), vr, dn_nn,
                        preferred_element_type=jnp.float32,
                    )
                    l = l1 + p2.sum(-1, keepdims=True)
                    om_ref[b, 2 * hh + i] = (o1 + o2) / l

        def run(mx):
            ox = 1 - mx
            ypeer = (mx, 1 - my_y, my_z)
            xtwin = (ox, my_y, my_z)

            pack(mx)
            ych = []
            for c in range(C):
                rows = pl.ds(c * ROWS, ROWS)
                r = pltpu.make_async_remote_copy(
                    src_ref=loc.at[mx, rows, :],
                    dst_ref=rcv.at[mx, rows, :],
                    send_sem=ysend.at[c],
                    recv_sem=yrecv.at[c],
                    device_id=ypeer,
                    device_id_type=pl.DeviceIdType.MESH,
                )
                r.start()
                ych.append(r)
            pack(ox)

            fch = [
                pltpu.make_async_remote_copy(
                    src_ref=rcv.at[mx, pl.ds(c * ROWS, ROWS), :],
                    dst_ref=rcv.at[mx, pl.ds(c * ROWS, ROWS), :],
                    send_sem=fsend.at[c],
                    recv_sem=frecv.at[c],
                    device_id=xtwin,
                    device_id_type=pl.DeviceIdType.MESH,
                )
                for c in range(C)
            ]

            units = [(b, hh) for b in range(B) for hh in range(H // 2)]
            part = {0: [], 1: []}
            for c in range(C):
                ych[c].wait_recv()
                fch[c].start()
                b, hh = units[c]
                part[b].extend(local_unit(b, hh))

            remote_batch(mx, part[mx])

            for c in range(C):
                fwd_in = pltpu.make_async_remote_copy(
                    src_ref=rcv.at[ox, pl.ds(c * ROWS, ROWS), :],
                    dst_ref=rcv.at[ox, pl.ds(c * ROWS, ROWS), :],
                    send_sem=ysend.at[c],
                    recv_sem=frecv.at[c],
                    device_id=xtwin,
                    device_id_type=pl.DeviceIdType.MESH,
                )
                fwd_in.wait_recv()
            remote_batch(ox, part[ox])

            for r in ych:
                r.wait_send()
            for r in fch:
                r.wait_send()

        @pl.when(my_x == 0)
        def _():
            run(0)

        @pl.when(my_x == 1)
        def _():
            run(1)

    return pl.pallas_call(
        body,
        out_shape=jax.ShapeDtypeStruct((B, H, SQ, D), jnp.float32),
        in_specs=[pl.BlockSpec(memory_space=pltpu.VMEM)] * 3,
        out_specs=pl.BlockSpec(memory_space=pltpu.VMEM),
        scratch_shapes=[
            pltpu.VMEM((B, SQ, 2 * HD), jnp.bfloat16),
            pltpu.VMEM((B, SQ, 2 * HD), jnp.bfloat16),
            pltpu.SemaphoreType.DMA((C,)),
            pltpu.SemaphoreType.DMA((C,)),
            pltpu.SemaphoreType.DMA((C,)),
            pltpu.SemaphoreType.DMA((C,)),
        ],
        compiler_params=pltpu.CompilerParams(collective_id=0),
    )(qt, km, vm)


def kernel(Q, K, V):
    km = K.reshape(B, SQ, HD)
    vm = V.reshape(B, SQ, HD)
    qt = jnp.transpose(Q, (0, 2, 1, 3))
    om = _fused(qt, km, vm)
    return jnp.transpose(om, (0, 2, 1, 3))
